# baseline (speedup 1.0000x reference)
import functools
import math

import jax
import jax.numpy as jnp
from jax import lax
from jax.experimental import pallas as pl
from jax.experimental.pallas import tpu as pltpu
from jax.experimental.pallas import tpu_sc as plsc

EMBED = 64
SCALE = 8.0
LANES = 16
NC = 2
NS = 16
NW = NC * NS
G = 128


@functools.lru_cache(maxsize=None)
def _build(n_total: int, vocab: int):
    assert n_total % (NW * G) == 0
    per_w = n_total // NW
    n_g = per_w // G

    mesh = plsc.VectorSubcoreMesh(core_axis_name="c", subcore_axis_name="s")

    @functools.partial(
        pl.kernel,
        mesh=mesh,
        out_type=jax.ShapeDtypeStruct((n_total, EMBED), jnp.float32),
        scratch_types=[
            pltpu.VMEM((n_g, G), jnp.int32),
            pltpu.VMEM((G, EMBED), jnp.float32),
            pltpu.SemaphoreType.DMA,
        ],
        compiler_params=pltpu.CompilerParams(use_tc_tiling_on_sc=False),
    )
    def emb_kernel(x_hbm, table_hbm, out_hbm, idx_v, rows_v, gsem):
        wid = lax.axis_index("s") * NC + lax.axis_index("c")
        pltpu.sync_copy(x_hbm.at[pl.ds(wid * n_g, n_g)], idx_v)

        def group(g, carry):
            pltpu.make_async_copy(
                table_hbm.at[idx_v.at[g]], rows_v, gsem
            ).start()
            pltpu.make_async_copy(
                table_hbm.at[idx_v.at[g]], rows_v, gsem
            ).wait()

            def row(r, c):
                for j in range(EMBED // LANES):
                    sl = pl.ds(j * LANES, LANES)
                    rows_v[r, sl] = rows_v[r, sl] * SCALE
                return c

            lax.fori_loop(0, G, row, 0, unroll=2)
            out_base = wid * per_w + g * G
            pltpu.sync_copy(rows_v, out_hbm.at[pl.ds(out_base, G)])
            return carry

        lax.fori_loop(0, n_g, group, 0)

    return emb_kernel


def kernel(x, table):
    b, h = x.shape
    n_total = b * h
    x_groups = x.reshape(n_total // G, G).astype(jnp.int32)
    out = _build(n_total, table.shape[0])(x_groups, table)
    return out.reshape(b, h, EMBED)


# speedup vs baseline: 1.0540x; 1.0540x over previous
import functools
import math

import jax
import jax.numpy as jnp
from jax import lax
from jax.experimental import pallas as pl
from jax.experimental.pallas import tpu as pltpu
from jax.experimental.pallas import tpu_sc as plsc

EMBED = 64
SCALE = 8.0
LANES = 16
NC = 2
NS = 16
NW = NC * NS
G = 128
NBUF = 4


@functools.lru_cache(maxsize=None)
def _build(n_total: int, vocab: int):
    assert n_total % (NW * G) == 0
    per_w = n_total // NW
    n_g = per_w // G
    assert n_g % NBUF == 0
    n_cyc = n_g // NBUF

    mesh = plsc.VectorSubcoreMesh(core_axis_name="c", subcore_axis_name="s")

    @functools.partial(
        pl.kernel,
        mesh=mesh,
        out_type=jax.ShapeDtypeStruct((n_total, EMBED), jnp.float32),
        scratch_types=[
            pltpu.VMEM((n_g, G), jnp.int32),
            pltpu.VMEM((NBUF, G, EMBED), jnp.float32),
            pltpu.VMEM((NBUF, G, EMBED), jnp.float32),
            pltpu.SemaphoreType.DMA((NBUF,)),
            pltpu.SemaphoreType.DMA((NBUF,)),
        ],
        compiler_params=pltpu.CompilerParams(use_tc_tiling_on_sc=False),
    )
    def emb_kernel(x_hbm, table_hbm, out_hbm, idx_v, gbuf, sbuf, gsem, ssem):
        wid = lax.axis_index("s") * NC + lax.axis_index("c")
        base = wid * per_w
        pltpu.sync_copy(x_hbm.at[pl.ds(wid * n_g, n_g)], idx_v)

        def start_gather(g, b):
            pltpu.make_async_copy(
                table_hbm.at[idx_v.at[g]], gbuf.at[b], gsem.at[b]
            ).start()

        def wait_gather(g, b):
            pltpu.make_async_copy(
                table_hbm.at[idx_v.at[g]], gbuf.at[b], gsem.at[b]
            ).wait()

        def start_scatter(g, b):
            pltpu.make_async_copy(
                sbuf.at[b], out_hbm.at[pl.ds(base + g * G, G)], ssem.at[b]
            ).start()

        def wait_scatter(g, b):
            pltpu.make_async_copy(
                sbuf.at[b], out_hbm.at[pl.ds(base + g * G, G)], ssem.at[b]
            ).wait()

        for b in range(NBUF):
            start_gather(b, b)

        def cycle(r, carry):
            for b in range(NBUF):
                g = r * NBUF + b
                wait_gather(g, b)

                @pl.when(r > 0)
                def _():
                    wait_scatter(g - NBUF, b)

                def row(rr, c):
                    for j in range(EMBED // LANES):
                        sl = pl.ds(j * LANES, LANES)
                        sbuf[b, rr, sl] = gbuf[b, rr, sl] * SCALE
                    return c

                lax.fori_loop(0, G, row, 0, unroll=2)
                start_scatter(g, b)

                @pl.when(r < n_cyc - 1)
                def _():
                    start_gather(g + NBUF, b)

            return carry

        lax.fori_loop(0, n_cyc, cycle, 0)

        for b in range(NBUF):
            wait_scatter(n_g - NBUF + b, b)

    return emb_kernel


def kernel(x, table):
    b, h = x.shape
    n_total = b * h
    x_groups = x.reshape(n_total // G, G).astype(jnp.int32)
    out = _build(n_total, table.shape[0])(x_groups, table)
    return out.reshape(b, h, EMBED)
